# Initial kernel scaffold; baseline (speedup 1.0000x reference)
import jax
import jax.numpy as jnp
from jax.experimental import pallas as pl


def kernel(elem_weights, elem_in_fea, batch, gw1, gb1, gw2, gb2, mw1, mb1, mw2, mb2, ln_g, ln_b):
    raise NotImplementedError("write your pallas kernel here")


# speedup vs baseline: 3.9492x; 3.9492x over previous
import functools

import jax
import jax.numpy as jnp
from jax.experimental import pallas as pl
from jax.experimental.pallas import tpu as pltpu

N = 50000
F = 256
G = 1024
R1 = 1000
NB1 = N // R1
RB = 1000
NB = N // RB

_SELU_A = 1.6732632423543772
_SELU_S = 1.0507009873554805
_NEG = -1e30


def _selu(x):
    return _SELU_S * jnp.where(x > 0, x, _SELU_A * (jnp.exp(x) - 1.0))


def _mlp_body(x_ref, gw1_ref, gb1_ref, gw2_ref, gb2_ref,
              mw1_ref, mb1_ref, mw2_ref, mb2_ref, gate_ref, msg_ref):
    x = x_ref[...]
    h = _selu(jnp.dot(x, gw1_ref[...], preferred_element_type=jnp.float32)
              + gb1_ref[...])
    g = (jnp.dot(h, gw2_ref[...], preferred_element_type=jnp.float32)
         + gb2_ref[...])
    gate_ref[...] = g.reshape(1, R1, 1)
    m = _selu(jnp.dot(x, mw1_ref[...], preferred_element_type=jnp.float32)
              + mb1_ref[...])
    msg_ref[...] = _selu(jnp.dot(m, mw2_ref[...],
                                 preferred_element_type=jnp.float32)
                         + mb2_ref[...])


def _segmax_body(batch_ref, gate_ref, gmax_ref, gmax_s):
    b = pl.program_id(0)

    @pl.when(b == 0)
    def _init():
        gmax_s[...] = jnp.full((1, G), _NEG, jnp.float32)

    ids = batch_ref[0]
    oh = ids == jax.lax.broadcasted_iota(jnp.int32, (RB, G), 1)
    vals = jnp.where(oh, gate_ref[0], _NEG)
    gmax_s[...] = jnp.maximum(gmax_s[...], jnp.max(vals, axis=0, keepdims=True))

    @pl.when(b == NB - 1)
    def _flush():
        gmax_ref[...] = gmax_s[...]


def _scatter_body(batch_ref, gate_ref, msg_ref, gmax_ref, ctx_ref, s1_s, s2_s):
    b = pl.program_id(0)

    @pl.when(b == 0)
    def _init():
        s1_s[...] = jnp.zeros((G, 1), jnp.float32)
        s2_s[...] = jnp.zeros((G, F), jnp.float32)

    ids = batch_ref[0]
    oh = ids == jax.lax.broadcasted_iota(jnp.int32, (RB, G), 1)
    ohf = oh.astype(jnp.float32)
    gmaxg = jnp.max(jnp.where(oh, gmax_ref[...], _NEG), axis=1, keepdims=True)
    e = jnp.exp(jnp.minimum(gate_ref[0] - gmaxg, 0.0))
    w = e * msg_ref[...]
    dn = (((0,), (0,)), ((), ()))
    s1_s[...] += jax.lax.dot_general(ohf, e, dn,
                                     preferred_element_type=jnp.float32)
    s2_s[...] += jax.lax.dot_general(ohf, w, dn,
                                     preferred_element_type=jnp.float32)

    @pl.when(b == NB - 1)
    def _flush():
        ctx_ref[...] = s2_s[...] / jnp.maximum(s1_s[...], 1e-30)


def _gather_ln_body(batch_ref, x_ref, ctx_ref, ln_g_ref, ln_b_ref, out_ref):
    ids = batch_ref[0]
    ohf = (ids == jax.lax.broadcasted_iota(jnp.int32, (RB, G), 1)
           ).astype(jnp.float32)
    gathered = jnp.dot(ohf, ctx_ref[...], preferred_element_type=jnp.float32)
    u = x_ref[...] + gathered
    mean = jnp.mean(u, axis=1, keepdims=True)
    d = u - mean
    var = jnp.mean(d * d, axis=1, keepdims=True)
    out_ref[...] = (d * jax.lax.rsqrt(var + 1e-5)) * ln_g_ref[...] + ln_b_ref[...]


def _whole(shape):
    return pl.BlockSpec(shape, lambda b: tuple(0 for _ in shape))


def kernel(elem_weights, elem_in_fea, batch, gw1, gb1, gw2, gb2,
           mw1, mb1, mw2, mb2, ln_g, ln_b):
    del elem_weights
    x = elem_in_fea
    batch3 = batch.astype(jnp.int32).reshape(NB, RB, 1)

    gate, msg = pl.pallas_call(
        _mlp_body,
        grid=(NB1,),
        in_specs=[
            pl.BlockSpec((R1, F), lambda b: (b, 0)),
            _whole((F, 256)), _whole((1, 256)),
            _whole((256, 1)), _whole((1, 1)),
            _whole((F, 256)), _whole((1, 256)),
            _whole((256, F)), _whole((1, F)),
        ],
        out_specs=[
            pl.BlockSpec((1, R1, 1), lambda b: (b, 0, 0)),
            pl.BlockSpec((R1, F), lambda b: (b, 0)),
        ],
        out_shape=[
            jax.ShapeDtypeStruct((NB1, R1, 1), jnp.float32),
            jax.ShapeDtypeStruct((N, F), jnp.float32),
        ],
        compiler_params=pltpu.CompilerParams(
            dimension_semantics=("parallel",)),
    )(x, gw1, gb1.reshape(1, -1), gw2, gb2.reshape(1, -1),
      mw1, mb1.reshape(1, -1), mw2, mb2.reshape(1, -1))
    gate3 = gate

    gmax = pl.pallas_call(
        _segmax_body,
        grid=(NB,),
        in_specs=[
            pl.BlockSpec((1, RB, 1), lambda b: (b, 0, 0)),
            pl.BlockSpec((1, RB, 1), lambda b: (b, 0, 0)),
        ],
        out_specs=_whole((1, G)),
        out_shape=jax.ShapeDtypeStruct((1, G), jnp.float32),
        scratch_shapes=[pltpu.VMEM((1, G), jnp.float32)],
        compiler_params=pltpu.CompilerParams(
            dimension_semantics=("arbitrary",)),
    )(batch3, gate3)

    ctx = pl.pallas_call(
        _scatter_body,
        grid=(NB,),
        in_specs=[
            pl.BlockSpec((1, RB, 1), lambda b: (b, 0, 0)),
            pl.BlockSpec((1, RB, 1), lambda b: (b, 0, 0)),
            pl.BlockSpec((RB, F), lambda b: (b, 0)),
            _whole((1, G)),
        ],
        out_specs=_whole((G, F)),
        out_shape=jax.ShapeDtypeStruct((G, F), jnp.float32),
        scratch_shapes=[pltpu.VMEM((G, 1), jnp.float32),
                        pltpu.VMEM((G, F), jnp.float32)],
        compiler_params=pltpu.CompilerParams(
            dimension_semantics=("arbitrary",)),
    )(batch3, gate3, msg, gmax)

    out = pl.pallas_call(
        _gather_ln_body,
        grid=(NB,),
        in_specs=[
            pl.BlockSpec((1, RB, 1), lambda b: (b, 0, 0)),
            pl.BlockSpec((RB, F), lambda b: (b, 0)),
            _whole((G, F)),
            _whole((1, F)),
            _whole((1, F)),
        ],
        out_specs=pl.BlockSpec((RB, F), lambda b: (b, 0)),
        out_shape=jax.ShapeDtypeStruct((N, F), jnp.float32),
        compiler_params=pltpu.CompilerParams(
            dimension_semantics=("parallel",)),
    )(batch3, x, ctx, ln_g.reshape(1, -1), ln_b.reshape(1, -1))

    return out
